# Initial kernel scaffold; baseline (speedup 1.0000x reference)
import jax
import jax.numpy as jnp
from jax.experimental import pallas as pl


def kernel(x, edge_index, edge_weight, W1, b1, W2, b2, gamma, beta):
    raise NotImplementedError("write your pallas kernel here")


# speedup vs baseline: 9.2254x; 9.2254x over previous
import functools

import jax
import jax.numpy as jnp
from jax import lax
from jax.experimental import pallas as pl
from jax.experimental.pallas import tpu as pltpu
from jax.experimental.pallas import tpu_sc as plsc

NC = 2
NS = 16
NW = NC * NS
LANES = 16
EPS = 1e-5


def _sc_mesh():
    return plsc.VectorSubcoreMesh(core_axis_name="c", subcore_axis_name="s")


def _pick_chunk(epw):
    for c in (128, 120, 112, 104, 96, 88, 80, 72, 64, 56, 48, 40, 32, 24, 16, 8):
        if epw % c == 0:
            return c
    return 8


@functools.lru_cache(maxsize=None)
def _make_deg(E, N):
    EPW = E // NW
    C = _pick_chunk(EPW)
    nchunks = EPW // C

    @functools.partial(
        pl.kernel,
        out_type=jax.ShapeDtypeStruct((NC, N), jnp.float32),
        mesh=_sc_mesh(),
        scratch_types=[
            pltpu.VMEM((C,), jnp.int32),
            pltpu.VMEM((C,), jnp.float32),
            pltpu.VMEM_SHARED((N,), jnp.float32),
        ],
    )
    def deg_kernel(dst_hbm, w_hbm, zeros_hbm, out_hbm, dstv, wv, acc):
        cid = lax.axis_index("c")
        sid = lax.axis_index("s")
        wid = cid * NS + sid

        @pl.when(sid == 0)
        def _():
            pltpu.sync_copy(zeros_hbm, acc)

        plsc.subcore_barrier()

        def body(k, carry):
            base = wid * EPW + k * C
            pltpu.sync_copy(dst_hbm.at[pl.ds(base, C)], dstv)
            pltpu.sync_copy(w_hbm.at[pl.ds(base, C)], wv)
            pltpu.sync_copy(wv, acc.at[dstv], add=True)
            return carry

        lax.fori_loop(0, nchunks, body, 0)
        plsc.subcore_barrier()

        @pl.when(sid == 0)
        def _():
            pltpu.sync_copy(acc, out_hbm.at[cid])

    return deg_kernel


@functools.lru_cache(maxsize=None)
def _make_agg(E, N, D):
    EPW = E // NW
    C = _pick_chunk(EPW)
    nchunks = EPW // C
    RPT = (N // NS) & ~7
    REM = N - NS * RPT

    @functools.partial(
        pl.kernel,
        out_type=jax.ShapeDtypeStruct((NC * N, D), jnp.float32),
        mesh=_sc_mesh(),
        scratch_types=[
            pltpu.VMEM((C,), jnp.int32),
            pltpu.VMEM((C,), jnp.int32),
            pltpu.VMEM((C,), jnp.float32),
            pltpu.VMEM((C, D), jnp.float32),
            pltpu.VMEM_SHARED((N, D), jnp.float32),
            pltpu.SemaphoreType.DMA,
        ],
    )
    def agg_kernel(src_hbm, dst_hbm, w_hbm, hs_hbm, zeros_hbm, out_hbm,
                   srcv, dstv, wv, rows, acc, sem):
        cid = lax.axis_index("c")
        sid = lax.axis_index("s")
        wid = cid * NS + sid

        pltpu.sync_copy(zeros_hbm.at[pl.ds(sid * RPT, RPT)],
                        acc.at[pl.ds(sid * RPT, RPT)])
        if REM:
            @pl.when(sid == NS - 1)
            def _():
                pltpu.sync_copy(zeros_hbm.at[pl.ds(NS * RPT, REM)],
                                acc.at[pl.ds(NS * RPT, REM)])
        plsc.subcore_barrier()

        def body(k, carry):
            base = wid * EPW + k * C
            pltpu.sync_copy(src_hbm.at[pl.ds(base, C)], srcv)
            pltpu.sync_copy(dst_hbm.at[pl.ds(base, C)], dstv)
            pltpu.sync_copy(w_hbm.at[pl.ds(base, C)], wv)
            pltpu.async_copy(hs_hbm.at[srcv], rows, sem).wait()

            def scale(g, c2):
                wv16 = wv[pl.ds(g * LANES, LANES)]
                for j in range(LANES):
                    w_s = wv16[j]
                    i = g * LANES + j
                    for k in range(D // LANES):
                        sl = pl.ds(k * LANES, LANES)
                        rows[i, sl] = rows[i, sl] * w_s
                return c2

            lax.fori_loop(0, C // LANES, scale, 0)
            pltpu.sync_copy(rows, acc.at[dstv], add=True)
            return carry

        lax.fori_loop(0, nchunks, body, 0)
        plsc.subcore_barrier()
        pltpu.sync_copy(acc.at[pl.ds(sid * RPT, RPT)],
                        out_hbm.at[pl.ds(cid * N + sid * RPT, RPT)])
        if REM:
            @pl.when(sid == NS - 1)
            def _():
                pltpu.sync_copy(acc.at[pl.ds(NS * RPT, REM)],
                                out_hbm.at[pl.ds(cid * N + NS * RPT, REM)])

    return agg_kernel


def _tc_pre(x, W1, degt, B):
    N, D = x.shape
    H = W1.shape[1]
    nblk = N // B

    def body(x_ref, w_ref, degt_ref, hs_ref, dis_ref):
        deg = degt_ref[:, 0:1] + degt_ref[:, 1:2] + 1.0
        dis = jnp.where(deg > 0, lax.rsqrt(jnp.maximum(deg, 1e-12)), 0.0)
        h = jnp.dot(x_ref[...], w_ref[...], preferred_element_type=jnp.float32)
        hs_ref[...] = h * dis
        dis_ref[...] = dis

    return pl.pallas_call(
        body,
        grid=(nblk,),
        in_specs=[
            pl.BlockSpec((B, D), lambda b: (b, 0)),
            pl.BlockSpec((D, H), lambda b: (0, 0)),
            pl.BlockSpec((B, NC), lambda b: (b, 0)),
        ],
        out_specs=[
            pl.BlockSpec((B, H), lambda b: (b, 0)),
            pl.BlockSpec((B, 1), lambda b: (b, 0)),
        ],
        out_shape=[
            jax.ShapeDtypeStruct((N, H), jnp.float32),
            jax.ShapeDtypeStruct((N, 1), jnp.float32),
        ],
    )(x, W1, degt)


def _tc_mid(accp, hs1, dis, b1, W2, B):
    N, H = hs1.shape
    D = W2.shape[1]
    nblk = N // B

    def body(a0_ref, a1_ref, hs1_ref, dis_ref, b1_ref, w2_ref, hs2_ref):
        a = (a0_ref[...] + a1_ref[...] + hs1_ref[...]) * dis_ref[...] + b1_ref[...]
        a = jnp.maximum(a, 0.0)
        h2 = jnp.dot(a, w2_ref[...], preferred_element_type=jnp.float32)
        hs2_ref[...] = h2 * dis_ref[...]

    return pl.pallas_call(
        body,
        grid=(nblk,),
        in_specs=[
            pl.BlockSpec((B, H), lambda b: (b, 0)),
            pl.BlockSpec((B, H), lambda b, _n=nblk: (b + _n, 0)),
            pl.BlockSpec((B, H), lambda b: (b, 0)),
            pl.BlockSpec((B, 1), lambda b: (b, 0)),
            pl.BlockSpec((1, H), lambda b: (0, 0)),
            pl.BlockSpec((H, D), lambda b: (0, 0)),
        ],
        out_specs=pl.BlockSpec((B, D), lambda b: (b, 0)),
        out_shape=jax.ShapeDtypeStruct((N, D), jnp.float32),
    )(accp, accp, hs1, dis, b1, W2)


def _tc_fin(accp, hs2, dis, b2, gamma, beta, B):
    N, D = hs2.shape
    nblk = N // B

    def body(a0_ref, a1_ref, hs2_ref, dis_ref, b2_ref, g_ref, be_ref, out_ref):
        a = (a0_ref[...] + a1_ref[...] + hs2_ref[...]) * dis_ref[...] + b2_ref[...]
        mu = jnp.mean(a, axis=1, keepdims=True)
        d = a - mu
        var = jnp.mean(d * d, axis=1, keepdims=True)
        out_ref[...] = d * lax.rsqrt(var + EPS) * g_ref[...] + be_ref[...]

    return pl.pallas_call(
        body,
        grid=(nblk,),
        in_specs=[
            pl.BlockSpec((B, D), lambda b: (b, 0)),
            pl.BlockSpec((B, D), lambda b, _n=nblk: (b + _n, 0)),
            pl.BlockSpec((B, D), lambda b: (b, 0)),
            pl.BlockSpec((B, 1), lambda b: (b, 0)),
            pl.BlockSpec((1, D), lambda b: (0, 0)),
            pl.BlockSpec((1, D), lambda b: (0, 0)),
            pl.BlockSpec((1, D), lambda b: (0, 0)),
        ],
        out_specs=pl.BlockSpec((B, D), lambda b: (b, 0)),
        out_shape=jax.ShapeDtypeStruct((N, D), jnp.float32),
    )(accp, accp, hs2, dis, b2, gamma, beta)


def kernel(x, edge_index, edge_weight, W1, b1, W2, b2, gamma, beta):
    N, D = x.shape
    H = W1.shape[1]
    E = edge_weight.shape[0]
    src = edge_index[0]
    dst = edge_index[1]
    w = edge_weight.astype(jnp.float32)

    zeros1 = jnp.zeros((N,), jnp.float32)
    zeros2 = jnp.zeros((N, max(D, H)), jnp.float32)
    b1r = b1.reshape(1, H)
    b2r = b2.reshape(1, D)
    gr = gamma.reshape(1, D)
    br = beta.reshape(1, D)

    B = 200 if N % 200 == 0 else 8

    degp = _make_deg(E, N)(dst, w, zeros1)
    degt = degp.T

    hs1, dis = _tc_pre(x, W1, degt, B)
    accp1 = _make_agg(E, N, H)(src, dst, w, hs1, zeros2[:, :H])
    hs2 = _tc_mid(accp1, hs1, dis, b1r, W2, B)
    accp2 = _make_agg(E, N, D)(src, dst, w, hs2, zeros2[:, :D])
    out = _tc_fin(accp2, hs2, dis, b2r, gr, br, B)
    return out


# speedup vs baseline: 21.0963x; 2.2868x over previous
import functools

import jax
import jax.numpy as jnp
from jax import lax
from jax.experimental import pallas as pl
from jax.experimental.pallas import tpu as pltpu
from jax.experimental.pallas import tpu_sc as plsc

NC = 2
NS = 16
NW = NC * NS
LANES = 16
EPS = 1e-5


def _sc_mesh():
    return plsc.VectorSubcoreMesh(core_axis_name="c", subcore_axis_name="s")


def _pick_chunk(epw):
    for c in (128, 120, 112, 104, 96, 88, 80, 72, 64, 56, 48, 40, 32, 24, 16, 8):
        if epw % c == 0:
            return c
    return 8


@functools.lru_cache(maxsize=None)
def _make_deg(E, N):
    EPW = E // NW
    C = _pick_chunk(EPW)
    NK = EPW // C
    RPT = (N // NS) & ~7
    REM = N - NS * RPT

    @functools.partial(
        pl.kernel,
        out_type=jax.ShapeDtypeStruct((NC, N), jnp.float32),
        mesh=_sc_mesh(),
        scratch_types=[
            pltpu.VMEM((NK, C), jnp.int32),
            pltpu.VMEM((NK, C), jnp.float32),
            pltpu.VMEM_SHARED((N,), jnp.float32),
            pltpu.SemaphoreType.DMA,
        ],
    )
    def deg_kernel(dst_hbm, w_hbm, zeros_hbm, out_hbm, dstv, wv, acc, sem):
        cid = lax.axis_index("c")
        sid = lax.axis_index("s")
        wid = cid * NS + sid

        pltpu.sync_copy(dst_hbm.at[wid], dstv)
        pltpu.sync_copy(w_hbm.at[wid], wv)

        @pl.when(sid == 0)
        def _():
            pltpu.sync_copy(zeros_hbm, acc)

        plsc.subcore_barrier()

        def fire(k, carry):
            pltpu.async_copy(wv.at[k], acc.at[dstv.at[k]], sem, add=True)
            return carry

        lax.fori_loop(0, NK, fire, 0)

        def drain(k, carry):
            pltpu.make_async_copy(wv.at[0], acc.at[dstv.at[0]], sem).wait()
            return carry

        lax.fori_loop(0, NK, drain, 0)
        plsc.subcore_barrier()

        @pl.when(sid == 0)
        def _():
            pltpu.sync_copy(acc, out_hbm.at[cid])

    return deg_kernel


@functools.lru_cache(maxsize=None)
def _make_agg(E, N, D):
    EPW = E // NW
    C = _pick_chunk(EPW)
    NK = EPW // C
    RPT = (N // NS) & ~7
    REM = N - NS * RPT

    assert NK >= 3 and NK % 2 == 1, NK

    @functools.partial(
        pl.kernel,
        out_type=jax.ShapeDtypeStruct((NC * N, D), jnp.float32),
        mesh=_sc_mesh(),
        scratch_types=[
            pltpu.VMEM((NK, C), jnp.int32),
            pltpu.VMEM((C,), jnp.int32),
            pltpu.VMEM((C,), jnp.int32),
            pltpu.VMEM((C,), jnp.float32),
            pltpu.VMEM((C,), jnp.float32),
            pltpu.VMEM((C, D), jnp.float32),
            pltpu.VMEM((C, D), jnp.float32),
            pltpu.VMEM_SHARED((N, D), jnp.float32),
            pltpu.SemaphoreType.DMA,
            pltpu.SemaphoreType.DMA,
            pltpu.SemaphoreType.DMA,
            pltpu.SemaphoreType.DMA,
            pltpu.SemaphoreType.DMA,
            pltpu.SemaphoreType.DMA,
        ],
    )
    def agg_kernel(src_hbm, dst3_hbm, w_hbm, hs_hbm, zeros_hbm, out_hbm,
                   dstv, srcb0, srcb1, wb0, wb1, rows0, rows1, acc,
                   gsem0, gsem1, ssem0, ssem1, msem0, msem1):
        cid = lax.axis_index("c")
        sid = lax.axis_index("s")
        wid = cid * NS + sid
        srcb = (srcb0, srcb1)
        wb = (wb0, wb1)
        rows = (rows0, rows1)
        gsem = (gsem0, gsem1)
        ssem = (ssem0, ssem1)
        msem = (msem0, msem1)

        pltpu.sync_copy(dst3_hbm.at[wid], dstv)
        pltpu.sync_copy(zeros_hbm.at[pl.ds(sid * RPT, RPT)],
                        acc.at[pl.ds(sid * RPT, RPT)])
        if REM:
            @pl.when(sid == NS - 1)
            def _():
                pltpu.sync_copy(zeros_hbm.at[pl.ds(NS * RPT, REM)],
                                acc.at[pl.ds(NS * RPT, REM)])
        plsc.subcore_barrier()

        def start_meta(k, b):
            base = wid * EPW + k * C
            pltpu.async_copy(src_hbm.at[pl.ds(base, C)], srcb[b], msem[b])
            pltpu.async_copy(w_hbm.at[pl.ds(base, C)], wb[b], msem[b])

        def wait_meta(b):
            pltpu.make_async_copy(src_hbm.at[pl.ds(0, C)], srcb[b], msem[b]).wait()
            pltpu.make_async_copy(w_hbm.at[pl.ds(0, C)], wb[b], msem[b]).wait()

        def start_gather(b):
            pltpu.async_copy(hs_hbm.at[srcb[b]], rows[b], gsem[b])

        def wait_gather(b):
            pltpu.make_async_copy(hs_hbm.at[srcb[b]], rows[b], gsem[b]).wait()

        def start_scatter(k, b):
            pltpu.async_copy(rows[b], acc.at[dstv.at[k]], ssem[b], add=True)

        def wait_scatter(b):
            pltpu.make_async_copy(rows[b], acc.at[dstv.at[0]], ssem[b]).wait()

        def scale(b):
            def body(g, c2):
                wv16 = wb[b][pl.ds(g * LANES, LANES)]
                for j in range(LANES):
                    w_s = wv16[j]
                    i = g * LANES + j
                    for q in range(D // LANES):
                        sl = pl.ds(q * LANES, LANES)
                        rows[b][i, sl] = rows[b][i, sl] * w_s
                return c2

            lax.fori_loop(0, C // LANES, body, 0)

        def step(k, b, first, last):
            wait_gather(b)
            if not last:
                wait_meta(1 - b)
            if not first:
                wait_scatter(1 - b)
            if not last:
                start_gather(1 - b)
            scale(b)
            start_scatter(k, b)
            if not last:
                if isinstance(k, int):
                    if k + 2 < NK:
                        start_meta(k + 2, b)
                else:
                    @pl.when(k + 2 < NK)
                    def _():
                        start_meta(k + 2, b)

        start_meta(0, 0)
        wait_meta(0)
        start_gather(0)
        start_meta(1, 1)

        step(0, 0, first=True, last=False)
        step(1, 1, first=False, last=False)

        def pair(j, carry):
            k0 = 2 * j
            step(k0, 0, first=False, last=False)
            step(k0 + 1, 1, first=False, last=False)
            return carry

        lax.fori_loop(1, (NK - 1) // 2, pair, 0)

        step(NK - 1, 0, first=False, last=True)
        wait_scatter(0)

        plsc.subcore_barrier()
        pltpu.sync_copy(acc.at[pl.ds(sid * RPT, RPT)],
                        out_hbm.at[pl.ds(cid * N + sid * RPT, RPT)])
        if REM:
            @pl.when(sid == NS - 1)
            def _():
                pltpu.sync_copy(acc.at[pl.ds(NS * RPT, REM)],
                                out_hbm.at[pl.ds(cid * N + NS * RPT, REM)])

    return agg_kernel


def _tc_pre(x, W1, degt, B):
    N, D = x.shape
    H = W1.shape[1]
    nblk = N // B

    def body(x_ref, w_ref, degt_ref, hs_ref, dis_ref):
        deg = degt_ref[:, 0:1] + degt_ref[:, 1:2] + 1.0
        dis = jnp.where(deg > 0, lax.rsqrt(jnp.maximum(deg, 1e-12)), 0.0)
        h = jnp.dot(x_ref[...], w_ref[...], preferred_element_type=jnp.float32)
        hs_ref[...] = h * dis
        dis_ref[...] = dis

    return pl.pallas_call(
        body,
        grid=(nblk,),
        in_specs=[
            pl.BlockSpec((B, D), lambda b: (b, 0)),
            pl.BlockSpec((D, H), lambda b: (0, 0)),
            pl.BlockSpec((B, NC), lambda b: (b, 0)),
        ],
        out_specs=[
            pl.BlockSpec((B, H), lambda b: (b, 0)),
            pl.BlockSpec((B, 1), lambda b: (b, 0)),
        ],
        out_shape=[
            jax.ShapeDtypeStruct((N, H), jnp.float32),
            jax.ShapeDtypeStruct((N, 1), jnp.float32),
        ],
    )(x, W1, degt)


def _tc_mid(accp, hs1, dis, b1, W2, B):
    N, H = hs1.shape
    D = W2.shape[1]
    nblk = N // B

    def body(a0_ref, a1_ref, hs1_ref, dis_ref, b1_ref, w2_ref, hs2_ref):
        a = (a0_ref[...] + a1_ref[...] + hs1_ref[...]) * dis_ref[...] + b1_ref[...]
        a = jnp.maximum(a, 0.0)
        h2 = jnp.dot(a, w2_ref[...], preferred_element_type=jnp.float32)
        hs2_ref[...] = h2 * dis_ref[...]

    return pl.pallas_call(
        body,
        grid=(nblk,),
        in_specs=[
            pl.BlockSpec((B, H), lambda b: (b, 0)),
            pl.BlockSpec((B, H), lambda b, _n=nblk: (b + _n, 0)),
            pl.BlockSpec((B, H), lambda b: (b, 0)),
            pl.BlockSpec((B, 1), lambda b: (b, 0)),
            pl.BlockSpec((1, H), lambda b: (0, 0)),
            pl.BlockSpec((H, D), lambda b: (0, 0)),
        ],
        out_specs=pl.BlockSpec((B, D), lambda b: (b, 0)),
        out_shape=jax.ShapeDtypeStruct((N, D), jnp.float32),
    )(accp, accp, hs1, dis, b1, W2)


def _tc_fin(accp, hs2, dis, b2, gamma, beta, B):
    N, D = hs2.shape
    nblk = N // B

    def body(a0_ref, a1_ref, hs2_ref, dis_ref, b2_ref, g_ref, be_ref, out_ref):
        a = (a0_ref[...] + a1_ref[...] + hs2_ref[...]) * dis_ref[...] + b2_ref[...]
        mu = jnp.mean(a, axis=1, keepdims=True)
        d = a - mu
        var = jnp.mean(d * d, axis=1, keepdims=True)
        out_ref[...] = d * lax.rsqrt(var + EPS) * g_ref[...] + be_ref[...]

    return pl.pallas_call(
        body,
        grid=(nblk,),
        in_specs=[
            pl.BlockSpec((B, D), lambda b: (b, 0)),
            pl.BlockSpec((B, D), lambda b, _n=nblk: (b + _n, 0)),
            pl.BlockSpec((B, D), lambda b: (b, 0)),
            pl.BlockSpec((B, 1), lambda b: (b, 0)),
            pl.BlockSpec((1, D), lambda b: (0, 0)),
            pl.BlockSpec((1, D), lambda b: (0, 0)),
            pl.BlockSpec((1, D), lambda b: (0, 0)),
        ],
        out_specs=pl.BlockSpec((B, D), lambda b: (b, 0)),
        out_shape=jax.ShapeDtypeStruct((N, D), jnp.float32),
    )(accp, accp, hs2, dis, b2, gamma, beta)


def kernel(x, edge_index, edge_weight, W1, b1, W2, b2, gamma, beta):
    N, D = x.shape
    H = W1.shape[1]
    E = edge_weight.shape[0]
    src = edge_index[0]
    dst = edge_index[1]
    w = edge_weight.astype(jnp.float32)

    zeros1 = jnp.zeros((N,), jnp.float32)
    zeros2 = jnp.zeros((N, max(D, H)), jnp.float32)
    b1r = b1.reshape(1, H)
    b2r = b2.reshape(1, D)
    gr = gamma.reshape(1, D)
    br = beta.reshape(1, D)

    B = 200 if N % 200 == 0 else 8

    EPW = E // NW
    C = _pick_chunk(EPW)
    NK = EPW // C
    dst3 = dst.reshape(NW, NK, C)
    w3 = w.reshape(NW, NK, C)

    degp = _make_deg(E, N)(dst3, w3, zeros1)
    degt = degp.T

    hs1, dis = _tc_pre(x, W1, degt, B)
    accp1 = _make_agg(E, N, H)(src, dst3, w, hs1, zeros2[:, :H])
    hs2 = _tc_mid(accp1, hs1, dis, b1r, W2, B)
    accp2 = _make_agg(E, N, D)(src, dst3, w, hs2, zeros2[:, :D])
    out = _tc_fin(accp2, hs2, dis, b2r, gr, br, B)
    return out
